# baseline (speedup 1.0000x reference)
import functools
import math

import jax
import jax.numpy as jnp
import numpy as np
from jax import lax
from jax.experimental import pallas as pl
from jax.experimental.pallas import tpu as pltpu
from jax.experimental.pallas import tpu_sc as plsc

_WINDOW_LO = -10000
_WINDOW_HI = 10000
_BINW = 25
_NFINAL = (_WINDOW_HI - _WINDOW_LO) // _BINW
_LEVEL_NBINS = (4, 20, 40, 100, 200, 400, 800)
_NLEVELS = len(_LEVEL_NBINS)
_NCLUST = 16
_SCALE = math.exp(math.log(1.5))
_RB = 8

_E_MATS = tuple(
    np.kron(np.eye(nb, dtype=np.float32),
            np.ones((1, _NFINAL // nb), np.float32))
    for nb in _LEVEL_NBINS
)

_SC_NC = 2
_SC_NS = 16
_NW = _SC_NC * _SC_NS
_CHUNK = 128
_GRP = 16


def _build_w(regions_oi, biases, deltas):
    n_ro = regions_oi.shape[0]
    n_steps = n_ro // _RB
    n_delta_elems = n_ro * _NCLUST * sum(_LEVEL_NBINS)
    kl_const = -float(n_delta_elems) * (math.log(_SCALE)
                                        + 0.5 * math.log(2.0 * math.pi))
    kl_scale = -0.5 / (_SCALE * _SCALE)

    def body(idx_ref, *refs):
        bias_hbm = refs[0:_NLEVELS]
        delta_hbm = refs[_NLEVELS:2 * _NLEVELS]
        e_refs = refs[2 * _NLEVELS:3 * _NLEVELS]
        w_out = refs[3 * _NLEVELS]
        kl_out = refs[3 * _NLEVELS + 1]
        d_scr = refs[3 * _NLEVELS + 2:4 * _NLEVELS + 2]
        b_scr = refs[4 * _NLEVELS + 2:5 * _NLEVELS + 2]
        sems = refs[5 * _NLEVELS + 2]
        i = pl.program_id(0)
        n = pl.num_programs(0)

        def issue(step, slot):
            for k in range(_RB):
                row = idx_ref[_RB * step + k]
                for l in range(_NLEVELS):
                    pltpu.make_async_copy(delta_hbm[l].at[row],
                                          d_scr[l].at[slot, k],
                                          sems.at[slot]).start()
                    pltpu.make_async_copy(bias_hbm[l].at[row],
                                          b_scr[l].at[slot, k],
                                          sems.at[slot]).start()

        @pl.when(i == 0)
        def _():
            issue(0, 0)

        @pl.when(i + 1 < n)
        def _():
            issue(i + 1, (i + 1) % 2)

        slot = i % 2
        for l in range(_NLEVELS):
            pltpu.make_async_copy(delta_hbm[l].at[pl.ds(0, _RB)],
                                  d_scr[l].at[slot], sems.at[slot]).wait()
            pltpu.make_async_copy(bias_hbm[l].at[pl.ds(0, _RB)],
                                  b_scr[l].at[slot], sems.at[slot]).wait()

        acc = jnp.zeros((_RB * _NCLUST, _NFINAL), jnp.float32)
        sumsq = jnp.float32(0.0)
        for l in range(_NLEVELS):
            nb = _LEVEL_NBINS[l]
            rows = d_scr[l][slot]
            brows = b_scr[l][slot]
            sumsq = sumsq + jnp.sum(rows * rows)
            pieces = [rows[:, c * nb:(c + 1) * nb] + brows
                      for c in range(_NCLUST)]
            xl = jnp.concatenate(pieces, axis=0)
            acc = acc + jnp.dot(xl.astype(jnp.bfloat16), e_refs[l][...],
                                preferred_element_type=jnp.float32)

        m = jnp.max(acc, axis=1, keepdims=True)
        z = jnp.sum(jnp.exp(acc - m), axis=1, keepdims=True)
        wn = acc - m - jnp.log(z) - math.log(float(_BINW))
        w_out[0] = wn.reshape(_NCLUST, _RB, _NFINAL)

        @pl.when(i == 0)
        def _():
            kl_out[0, 0] = 0.0

        kl_out[0, 0] += sumsq

        @pl.when(i == pl.num_programs(0) - 1)
        def _():
            kl_out[0, 0] = kl_out[0, 0] * kl_scale + kl_const

    grid_spec = pltpu.PrefetchScalarGridSpec(
        num_scalar_prefetch=1,
        grid=(n_steps,),
        in_specs=[
            *[pl.BlockSpec(memory_space=pl.ANY) for _ in range(_NLEVELS)],
            *[pl.BlockSpec(memory_space=pl.ANY) for _ in range(_NLEVELS)],
            *[pl.BlockSpec((nb, _NFINAL), lambda i, idx: (0, 0))
              for nb in _LEVEL_NBINS],
        ],
        out_specs=[
            pl.BlockSpec((1, _NCLUST, _RB, _NFINAL),
                         lambda i, idx: (i, 0, 0, 0)),
            pl.BlockSpec((1, 1), lambda i, idx: (0, 0),
                         memory_space=pltpu.SMEM),
        ],
        scratch_shapes=[
            *[pltpu.VMEM((2, _RB, _NCLUST * nb), jnp.float32)
              for nb in _LEVEL_NBINS],
            *[pltpu.VMEM((2, _RB, nb), jnp.float32)
              for nb in _LEVEL_NBINS],
            pltpu.SemaphoreType.DMA((2,)),
        ],
    )
    return pl.pallas_call(
        body,
        grid_spec=grid_spec,
        out_shape=(
            jax.ShapeDtypeStruct((n_ro // _RB, _NCLUST, _RB, _NFINAL),
                                 jnp.float32),
            jax.ShapeDtypeStruct((1, 1), jnp.float32),
        ),
    )(regions_oi, *biases, *deltas,
      *[jnp.asarray(e, dtype=jnp.bfloat16) for e in _E_MATS])


def _fire_drain(fire_one, n, dummy_src, drain_dst, sem):
    n_grp = n // _GRP
    rem = n - n_grp * _GRP

    def grp(g, carry):
        for b in range(_GRP):
            fire_one(g * _GRP + b)
        pltpu.make_async_copy(dummy_src.at[pl.ds(0, _GRP)],
                              drain_dst.at[pl.ds(0, _GRP)], sem).wait()
        return carry

    lax.fori_loop(0, n_grp, grp, 0)
    if rem:
        for b in range(rem):
            fire_one(n_grp * _GRP + b)
        left = rem
        while left:
            q = 8 if left >= 8 else left
            pltpu.make_async_copy(dummy_src.at[pl.ds(0, q)],
                                  drain_dst.at[pl.ds(0, q)], sem).wait()
            left -= q


def _make_tc_prep(nchunk, n_cl_pad):
    def body(cut_ref, cl_ref, part_ref, cl6_ref):
        co = cut_ref[0]
        lr = cut_ref[1]
        co = jnp.minimum(jnp.maximum(co, _WINDOW_LO),
                         _WINDOW_HI - 1) - _WINDOW_LO
        b = lax.shift_right_logical(co * 5243, 17)
        hi = lax.shift_right_logical(lr, 3)
        part_ref[...] = (hi * (_NCLUST * _RB * _NFINAL)
                         + (lr - hi * _RB) * _NFINAL + b)
        cl6_ref[...] = cl_ref[...] * (_RB * _NFINAL)

    return pl.pallas_call(
        body,
        out_shape=(
            jax.ShapeDtypeStruct((_NW, nchunk, _CHUNK), jnp.int32),
            jax.ShapeDtypeStruct((n_cl_pad,), jnp.int32),
        ),
    )


def _make_sc_index(nchunk):
    mesh = plsc.VectorSubcoreMesh(core_axis_name="c", subcore_axis_name="s")

    def body(part_hbm, lc_hbm, cl6_hbm, idx_hbm, lc_v, cv_v, idx_v, sem):
        c = lax.axis_index("c")
        s = lax.axis_index("s")
        wid = s * _SC_NC + c
        pltpu.sync_copy(part_hbm.at[wid], idx_v)
        pltpu.sync_copy(lc_hbm.at[wid], lc_v)

        def fire_cl(j):
            pltpu.async_copy(cl6_hbm.at[lc_v.at[j]], cv_v.at[j], sem)

        _fire_drain(fire_cl, nchunk, part_hbm.at[wid], cv_v, sem)

        def add_body(j, carry):
            for k in range(_CHUNK // 16):
                sl = pl.ds(k * 16, 16)
                idx_v[j, sl] += cv_v[j, sl]
            return carry

        lax.fori_loop(0, nchunk, add_body, 0)

        pltpu.sync_copy(idx_v, idx_hbm.at[wid])

    return pl.kernel(
        body,
        out_type=jax.ShapeDtypeStruct((_NW, nchunk, _CHUNK), jnp.int32),
        mesh=mesh,
        scratch_types=[
            pltpu.VMEM((nchunk, _CHUNK), jnp.int32),
            pltpu.VMEM((nchunk, _CHUNK), jnp.int32),
            pltpu.VMEM((nchunk, _CHUNK), jnp.int32),
            pltpu.SemaphoreType.DMA,
        ],
    )


def _make_sc_gather(nchunk):
    mesh = plsc.VectorSubcoreMesh(core_axis_name="c", subcore_axis_name="s")

    def body(w_hbm, idx_hbm, out_hbm, idx_v, val_v, sem):
        c = lax.axis_index("c")
        s = lax.axis_index("s")
        wid = s * _SC_NC + c
        pltpu.sync_copy(idx_hbm.at[wid], idx_v)

        def fire_w(j):
            pltpu.async_copy(w_hbm.at[idx_v.at[j]], val_v.at[j], sem)

        _fire_drain(fire_w, nchunk, out_hbm.at[wid], val_v, sem)

        pltpu.sync_copy(val_v, out_hbm.at[wid])

    return pl.kernel(
        body,
        out_type=jax.ShapeDtypeStruct((_NW, nchunk, _CHUNK), jnp.float32),
        mesh=mesh,
        scratch_types=[
            pltpu.VMEM((nchunk, _CHUNK), jnp.int32),
            pltpu.VMEM((nchunk, _CHUNK), jnp.float32),
            pltpu.SemaphoreType.DMA,
        ],
    )


def kernel(regions_oi, coordinates, local_region_ix, local_cell_ix,
           clustering_indices, w_0, w_1, w_2, w_3, w_4, w_5, w_6,
           w_delta_0, w_delta_1, w_delta_2, w_delta_3, w_delta_4,
           w_delta_5, w_delta_6):
    ws = (w_0, w_1, w_2, w_3, w_4, w_5, w_6)
    wds = (w_delta_0, w_delta_1, w_delta_2, w_delta_3, w_delta_4,
           w_delta_5, w_delta_6)
    n_reg = ws[0].shape[0]
    biases = list(ws)
    deltas = list(wds)

    n_cuts = coordinates.shape[0]
    nchunk = -(-n_cuts // (_NW * _CHUNK))
    n_pad = _NW * nchunk * _CHUNK
    n_cl = clustering_indices.shape[0]
    n_clust_pad = -(-n_cl // 16) * 16

    cuts = jnp.stack([coordinates, local_region_ix, local_cell_ix])
    cuts = jnp.pad(cuts, ((0, 0), (0, n_pad - n_cuts)))
    cuts = cuts.reshape(3, _NW, nchunk, _CHUNK)
    cl_p = jnp.pad(clustering_indices, (0, n_clust_pad - n_cl))

    partial, cl6 = _make_tc_prep(nchunk, n_clust_pad)(cuts, cl_p)
    idx = _make_sc_index(nchunk)(partial, cuts[2], cl6)
    w_norm, kl = _build_w(regions_oi, biases, deltas)
    lik = _make_sc_gather(nchunk)(w_norm.reshape(-1), idx)
    return lik.reshape(-1)[:n_cuts], kl[0, 0]


# speedup vs baseline: 1.0896x; 1.0896x over previous
import functools
import math

import jax
import jax.numpy as jnp
import numpy as np
from jax import lax
from jax.experimental import pallas as pl
from jax.experimental.pallas import tpu as pltpu
from jax.experimental.pallas import tpu_sc as plsc

_WINDOW_LO = -10000
_WINDOW_HI = 10000
_BINW = 25
_NFINAL = (_WINDOW_HI - _WINDOW_LO) // _BINW
_LEVEL_NBINS = (4, 20, 40, 100, 200, 400, 800)
_NLEVELS = len(_LEVEL_NBINS)
_NCLUST = 16
_SCALE = math.exp(math.log(1.5))
_RB = 8

_E_MATS = tuple(
    np.kron(np.eye(nb, dtype=np.float32),
            np.ones((1, _NFINAL // nb), np.float32))
    for nb in _LEVEL_NBINS
)

_SC_NC = 2
_SC_NS = 16
_NW = _SC_NC * _SC_NS
_CHUNK = 128
_GRP = 16


def _build_w(regions_oi, biases, deltas):
    n_ro = regions_oi.shape[0]
    n_steps = n_ro // _RB
    n_delta_elems = n_ro * _NCLUST * sum(_LEVEL_NBINS)
    kl_const = -float(n_delta_elems) * (math.log(_SCALE)
                                        + 0.5 * math.log(2.0 * math.pi))
    kl_scale = -0.5 / (_SCALE * _SCALE)

    def body(idx_ref, *refs):
        bias_hbm = refs[0:_NLEVELS]
        delta_hbm = refs[_NLEVELS:2 * _NLEVELS]
        e_refs = refs[2 * _NLEVELS:3 * _NLEVELS]
        w_out = refs[3 * _NLEVELS]
        kl_out = refs[3 * _NLEVELS + 1]
        d_scr = refs[3 * _NLEVELS + 2:4 * _NLEVELS + 2]
        b_scr = refs[4 * _NLEVELS + 2:5 * _NLEVELS + 2]
        sems = refs[5 * _NLEVELS + 2]
        i = pl.program_id(0)
        n = pl.num_programs(0)

        def issue(step, slot):
            for k in range(_RB):
                row = idx_ref[_RB * step + k]
                for l in range(_NLEVELS):
                    pltpu.make_async_copy(delta_hbm[l].at[row],
                                          d_scr[l].at[slot, k],
                                          sems.at[slot]).start()
                    pltpu.make_async_copy(bias_hbm[l].at[row],
                                          b_scr[l].at[slot, k],
                                          sems.at[slot]).start()

        @pl.when(i == 0)
        def _():
            issue(0, 0)

        @pl.when(i + 1 < n)
        def _():
            issue(i + 1, (i + 1) % 2)

        slot = i % 2
        for l in range(_NLEVELS):
            pltpu.make_async_copy(delta_hbm[l].at[pl.ds(0, _RB)],
                                  d_scr[l].at[slot], sems.at[slot]).wait()
            pltpu.make_async_copy(bias_hbm[l].at[pl.ds(0, _RB)],
                                  b_scr[l].at[slot], sems.at[slot]).wait()

        acc = jnp.zeros((_RB * _NCLUST, _NFINAL), jnp.float32)
        sumsq = jnp.float32(0.0)
        for l in range(_NLEVELS):
            nb = _LEVEL_NBINS[l]
            rows = d_scr[l][slot]
            brows = b_scr[l][slot]
            sumsq = sumsq + jnp.sum(rows * rows)
            pieces = [rows[:, c * nb:(c + 1) * nb] + brows
                      for c in range(_NCLUST)]
            xl = jnp.concatenate(pieces, axis=0)
            acc = acc + jnp.dot(xl.astype(jnp.bfloat16), e_refs[l][...],
                                preferred_element_type=jnp.float32)

        m = jnp.max(acc, axis=1, keepdims=True)
        z = jnp.sum(jnp.exp(acc - m), axis=1, keepdims=True)
        wn = acc - m - jnp.log(z) - math.log(float(_BINW))
        w_out[0] = wn.reshape(_NCLUST, _RB, _NFINAL)

        @pl.when(i == 0)
        def _():
            kl_out[0, 0] = 0.0

        kl_out[0, 0] += sumsq

        @pl.when(i == pl.num_programs(0) - 1)
        def _():
            kl_out[0, 0] = kl_out[0, 0] * kl_scale + kl_const

    grid_spec = pltpu.PrefetchScalarGridSpec(
        num_scalar_prefetch=1,
        grid=(n_steps,),
        in_specs=[
            *[pl.BlockSpec(memory_space=pl.ANY) for _ in range(_NLEVELS)],
            *[pl.BlockSpec(memory_space=pl.ANY) for _ in range(_NLEVELS)],
            *[pl.BlockSpec((nb, _NFINAL), lambda i, idx: (0, 0))
              for nb in _LEVEL_NBINS],
        ],
        out_specs=[
            pl.BlockSpec((1, _NCLUST, _RB, _NFINAL),
                         lambda i, idx: (i, 0, 0, 0)),
            pl.BlockSpec((1, 1), lambda i, idx: (0, 0),
                         memory_space=pltpu.SMEM),
        ],
        scratch_shapes=[
            *[pltpu.VMEM((2, _RB, _NCLUST * nb), jnp.float32)
              for nb in _LEVEL_NBINS],
            *[pltpu.VMEM((2, _RB, nb), jnp.float32)
              for nb in _LEVEL_NBINS],
            pltpu.SemaphoreType.DMA((2,)),
        ],
    )
    return pl.pallas_call(
        body,
        grid_spec=grid_spec,
        out_shape=(
            jax.ShapeDtypeStruct((n_ro // _RB, _NCLUST, _RB, _NFINAL),
                                 jnp.float32),
            jax.ShapeDtypeStruct((1, 1), jnp.float32),
        ),
    )(regions_oi, *biases, *deltas,
      *[jnp.asarray(e, dtype=jnp.bfloat16) for e in _E_MATS])


def _fire_drain(fire_one, n, dummy_src, drain_dst, sem):
    n_grp = n // _GRP
    rem = n - n_grp * _GRP

    def grp(g, carry):
        for b in range(_GRP):
            fire_one(g * _GRP + b)
        pltpu.make_async_copy(dummy_src.at[pl.ds(0, _GRP)],
                              drain_dst.at[pl.ds(0, _GRP)], sem).wait()
        return carry

    lax.fori_loop(0, n_grp, grp, 0)
    if rem:
        for b in range(rem):
            fire_one(n_grp * _GRP + b)
        left = rem
        while left:
            q = 8 if left >= 8 else left
            pltpu.make_async_copy(dummy_src.at[pl.ds(0, q)],
                                  drain_dst.at[pl.ds(0, q)], sem).wait()
            left -= q


def _make_sc_index(nchunk):
    mesh = plsc.VectorSubcoreMesh(core_axis_name="c", subcore_axis_name="s")

    def body(cut_hbm, cl_hbm, idx_hbm, co_v, lr_v, lc_v, cv_v, idx_v, sem):
        c = lax.axis_index("c")
        s = lax.axis_index("s")
        wid = s * _SC_NC + c
        pltpu.sync_copy(cut_hbm.at[0, wid], co_v)
        pltpu.sync_copy(cut_hbm.at[1, wid], lr_v)
        pltpu.sync_copy(cut_hbm.at[2, wid], lc_v)

        def fire_cl(j):
            pltpu.async_copy(cl_hbm.at[lc_v.at[j]], cv_v.at[j], sem)

        _fire_drain(fire_cl, nchunk, cut_hbm.at[0, wid], cv_v, sem)

        def idx_body(j, carry):
            for k in range(_CHUNK // 16):
                sl = pl.ds(k * 16, 16)
                c16 = cv_v[j, sl]
                co16 = co_v[j, sl]
                co16 = jnp.minimum(jnp.maximum(co16, _WINDOW_LO),
                                   _WINDOW_HI - 1) - _WINDOW_LO
                b16 = lax.shift_right_logical(co16 * 5243, 17)
                lr16 = lr_v[j, sl]
                hi = lax.shift_right_logical(lr16, 3)
                lo = jnp.bitwise_and(lr16, 7)
                idx_v[j, sl] = (hi * (_NCLUST * _RB * _NFINAL)
                                + c16 * (_RB * _NFINAL)
                                + lo * _NFINAL + b16)
            return carry

        lax.fori_loop(0, nchunk, idx_body, 0)

        pltpu.sync_copy(idx_v, idx_hbm.at[wid])

    return pl.kernel(
        body,
        out_type=jax.ShapeDtypeStruct((_NW, nchunk, _CHUNK), jnp.int32),
        mesh=mesh,
        scratch_types=[
            pltpu.VMEM((nchunk, _CHUNK), jnp.int32),
            pltpu.VMEM((nchunk, _CHUNK), jnp.int32),
            pltpu.VMEM((nchunk, _CHUNK), jnp.int32),
            pltpu.VMEM((nchunk, _CHUNK), jnp.int32),
            pltpu.VMEM((nchunk, _CHUNK), jnp.int32),
            pltpu.SemaphoreType.DMA,
        ],
    )


def _make_sc_gather(nchunk):
    mesh = plsc.VectorSubcoreMesh(core_axis_name="c", subcore_axis_name="s")

    def body(w_hbm, idx_hbm, out_hbm, idx_v, val_v, sem):
        c = lax.axis_index("c")
        s = lax.axis_index("s")
        wid = s * _SC_NC + c
        pltpu.sync_copy(idx_hbm.at[wid], idx_v)

        def fire_w(j):
            pltpu.async_copy(w_hbm.at[idx_v.at[j]], val_v.at[j], sem)

        _fire_drain(fire_w, nchunk, out_hbm.at[wid], val_v, sem)

        pltpu.sync_copy(val_v, out_hbm.at[wid])

    return pl.kernel(
        body,
        out_type=jax.ShapeDtypeStruct((_NW, nchunk, _CHUNK), jnp.float32),
        mesh=mesh,
        scratch_types=[
            pltpu.VMEM((nchunk, _CHUNK), jnp.int32),
            pltpu.VMEM((nchunk, _CHUNK), jnp.float32),
            pltpu.SemaphoreType.DMA,
        ],
    )


def kernel(regions_oi, coordinates, local_region_ix, local_cell_ix,
           clustering_indices, w_0, w_1, w_2, w_3, w_4, w_5, w_6,
           w_delta_0, w_delta_1, w_delta_2, w_delta_3, w_delta_4,
           w_delta_5, w_delta_6):
    ws = (w_0, w_1, w_2, w_3, w_4, w_5, w_6)
    wds = (w_delta_0, w_delta_1, w_delta_2, w_delta_3, w_delta_4,
           w_delta_5, w_delta_6)
    n_reg = ws[0].shape[0]
    biases = list(ws)
    deltas = list(wds)

    n_cuts = coordinates.shape[0]
    nchunk = -(-n_cuts // (_NW * _CHUNK))
    n_pad = _NW * nchunk * _CHUNK
    n_cl = clustering_indices.shape[0]
    n_clust_pad = -(-n_cl // 16) * 16

    cuts = jnp.stack([coordinates, local_region_ix, local_cell_ix])
    cuts = jnp.pad(cuts, ((0, 0), (0, n_pad - n_cuts)))
    cuts = cuts.reshape(3, _NW, nchunk, _CHUNK)
    cl_p = jnp.pad(clustering_indices, (0, n_clust_pad - n_cl))

    idx = _make_sc_index(nchunk)(cuts, cl_p)
    w_norm, kl = _build_w(regions_oi, biases, deltas)
    lik = _make_sc_gather(nchunk)(w_norm.reshape(-1), idx)
    return lik.reshape(-1)[:n_cuts], kl[0, 0]
